# Initial kernel scaffold; baseline (speedup 1.0000x reference)
import jax
import jax.numpy as jnp
from jax.experimental import pallas as pl


def kernel(node_features, edge_index, weight0, bias0, weight1, bias1, hidden_dim):
    raise NotImplementedError("write your pallas kernel here")


# speedup vs baseline: 5.0866x; 5.0866x over previous
import functools

import jax
import jax.numpy as jnp
from jax import lax
from jax.experimental import pallas as pl
from jax.experimental.pallas import tpu as pltpu
from jax.experimental.pallas import tpu_sc as plsc

_NC = 2
_NS = 16
_CHUNK = 80


def _make_sc_aggregate(N, D, E):
    NW = _NC * _NS
    ep = E // NW
    nch = ep // _CHUNK
    rp = (N // (8 * _NS)) * 8
    tail = N - _NS * rp

    mesh = plsc.VectorSubcoreMesh(core_axis_name="c", subcore_axis_name="s")

    @functools.partial(
        pl.kernel,
        mesh=mesh,
        out_type=jax.ShapeDtypeStruct((_NC, N, D), jnp.float32),
        scratch_types=[
            pltpu.VMEM((_CHUNK,), jnp.int32),
            pltpu.VMEM((_CHUNK,), jnp.int32),
            pltpu.VMEM((_CHUNK, D), jnp.float32),
            pltpu.VMEM_SHARED((N, D), jnp.float32),
            pltpu.SemaphoreType.DMA,
        ],
    )
    def agg(t_hbm, row_hbm, col_hbm, zero_hbm, out_hbm,
            row_v, col_v, msg_v, agg_sh, sem):
        cid = lax.axis_index("c")
        sid = lax.axis_index("s")
        wid = cid * _NS + sid

        r0 = sid * rp
        pltpu.sync_copy(zero_hbm.at[pl.ds(r0, rp)], agg_sh.at[pl.ds(r0, rp)])
        if tail:
            @pl.when(sid == 0)
            def _():
                pltpu.sync_copy(zero_hbm.at[pl.ds(_NS * rp, tail)],
                                agg_sh.at[pl.ds(_NS * rp, tail)])
        plsc.subcore_barrier()

        def body(i, carry):
            base = wid * ep + i * _CHUNK
            pltpu.sync_copy(row_hbm.at[pl.ds(base, _CHUNK)], row_v)
            pltpu.async_copy(t_hbm.at[row_v], msg_v, sem).wait()
            pltpu.sync_copy(col_hbm.at[pl.ds(base, _CHUNK)], col_v)
            pltpu.sync_copy(msg_v, agg_sh.at[col_v], add=True)
            return carry

        lax.fori_loop(0, nch, body, 0)
        plsc.subcore_barrier()
        pltpu.sync_copy(agg_sh.at[pl.ds(r0, rp)],
                        out_hbm.at[cid, pl.ds(r0, rp)])
        if tail:
            @pl.when(sid == 0)
            def _():
                pltpu.sync_copy(agg_sh.at[pl.ds(_NS * rp, tail)],
                                out_hbm.at[cid, pl.ds(_NS * rp, tail)])

    return agg


_BR = 1000


def _tc_linear(x, w, b):
    N, D = x.shape

    def body(x_ref, w_ref, b_ref, o_ref):
        o_ref[...] = lax.dot_general(
            x_ref[...], w_ref[...], (((1,), (1,)), ((), ())),
            preferred_element_type=jnp.float32) + b_ref[...]

    return pl.pallas_call(
        body,
        grid=(N // _BR,),
        in_specs=[
            pl.BlockSpec((_BR, D), lambda i: (i, 0)),
            pl.BlockSpec((D, D), lambda i: (0, 0)),
            pl.BlockSpec((1, D), lambda i: (0, 0)),
        ],
        out_specs=pl.BlockSpec((_BR, D), lambda i: (i, 0)),
        out_shape=jax.ShapeDtypeStruct((N, D), jnp.float32),
    )(x, w, b)


def _tc_combine_linear(t, p, w, b):
    N, D = t.shape

    def body(t_ref, p_ref, w_ref, b_ref, o_ref):
        h = (t_ref[...] + p_ref[0] + p_ref[1]) * 0.5
        h = jnp.maximum(h, 0.0)
        o_ref[...] = lax.dot_general(
            h, w_ref[...], (((1,), (1,)), ((), ())),
            preferred_element_type=jnp.float32) + b_ref[...]

    return pl.pallas_call(
        body,
        grid=(N // _BR,),
        in_specs=[
            pl.BlockSpec((_BR, D), lambda i: (i, 0)),
            pl.BlockSpec((_NC, _BR, D), lambda i: (0, i, 0)),
            pl.BlockSpec((D, D), lambda i: (0, 0)),
            pl.BlockSpec((1, D), lambda i: (0, 0)),
        ],
        out_specs=pl.BlockSpec((_BR, D), lambda i: (i, 0)),
        out_shape=jax.ShapeDtypeStruct((N, D), jnp.float32),
    )(t, p, w, b)


def _tc_combine(t, p):
    N, D = t.shape

    def body(t_ref, p_ref, o_ref):
        o_ref[...] = (t_ref[...] + p_ref[0] + p_ref[1]) * 0.5

    return pl.pallas_call(
        body,
        grid=(N // _BR,),
        in_specs=[
            pl.BlockSpec((_BR, D), lambda i: (i, 0)),
            pl.BlockSpec((_NC, _BR, D), lambda i: (0, i, 0)),
        ],
        out_specs=pl.BlockSpec((_BR, D), lambda i: (i, 0)),
        out_shape=jax.ShapeDtypeStruct((N, D), jnp.float32),
    )(t, p)


def kernel(node_features, edge_index, weight0, bias0, weight1, bias1,
           hidden_dim):
    N, D = node_features.shape
    E = edge_index.shape[1]
    row = edge_index[0]
    col = edge_index[1]
    zeros = jnp.zeros((N, D), jnp.float32)

    sc_aggregate = _make_sc_aggregate(N, D, E)

    t1 = _tc_linear(node_features, weight0[0], bias0)
    p1 = sc_aggregate(t1, row, col, zeros)
    t2 = _tc_combine_linear(t1, p1, weight1[0], bias1)
    p2 = sc_aggregate(t2, row, col, zeros)
    return _tc_combine(t2, p2)


# speedup vs baseline: 11.5442x; 2.2695x over previous
import functools

import jax
import jax.numpy as jnp
from jax import lax
from jax.experimental import pallas as pl
from jax.experimental.pallas import tpu as pltpu
from jax.experimental.pallas import tpu_sc as plsc

_NC = 2
_NS = 16
_CHUNK = 80


def _make_sc_aggregate(N, D, E):
    NW = _NC * _NS
    ep = E // NW
    nch = ep // _CHUNK
    rp = (N // (8 * _NS)) * 8
    tail = N - _NS * rp

    mesh = plsc.VectorSubcoreMesh(core_axis_name="c", subcore_axis_name="s")

    @functools.partial(
        pl.kernel,
        mesh=mesh,
        out_type=jax.ShapeDtypeStruct((_NC, N, D), jnp.float32),
        scratch_types=[
            pltpu.VMEM((ep,), jnp.int32),
            pltpu.VMEM((_CHUNK,), jnp.int32),
            pltpu.VMEM((_CHUNK,), jnp.int32),
            pltpu.VMEM((_CHUNK, D), jnp.float32),
            pltpu.VMEM((_CHUNK, D), jnp.float32),
            pltpu.VMEM_SHARED((N, D), jnp.float32),
            pltpu.SemaphoreType.DMA,
            pltpu.SemaphoreType.DMA,
            pltpu.SemaphoreType.DMA,
            pltpu.SemaphoreType.DMA,
        ],
    )
    def agg(t_hbm, row_hbm, col_hbm, zero_hbm, out_hbm,
            row_all, col0, col1, msg0, msg1, agg_sh,
            gsem0, gsem1, csem0, csem1):
        cid = lax.axis_index("c")
        sid = lax.axis_index("s")
        wid = cid * _NS + sid
        e0 = wid * ep
        cols = (col0, col1)
        msgs = (msg0, msg1)
        gsems = (gsem0, gsem1)
        csems = (csem0, csem1)

        r0 = sid * rp
        pltpu.sync_copy(zero_hbm.at[pl.ds(r0, rp)], agg_sh.at[pl.ds(r0, rp)])
        if tail:
            @pl.when(sid == 0)
            def _():
                pltpu.sync_copy(zero_hbm.at[pl.ds(_NS * rp, tail)],
                                agg_sh.at[pl.ds(_NS * rp, tail)])
        pltpu.sync_copy(row_hbm.at[pl.ds(e0, ep)], row_all)
        plsc.subcore_barrier()

        def start(j, b):
            pltpu.async_copy(col_hbm.at[pl.ds(e0 + j * _CHUNK, _CHUNK)],
                             cols[b], csems[b])
            pltpu.async_copy(t_hbm.at[row_all.at[pl.ds(j * _CHUNK, _CHUNK)]],
                             msgs[b], gsems[b])

        def finish(j, b):
            pltpu.make_async_copy(col_hbm.at[pl.ds(e0 + j * _CHUNK, _CHUNK)],
                                  cols[b], csems[b]).wait()
            pltpu.make_async_copy(t_hbm.at[row_all.at[pl.ds(j * _CHUNK, _CHUNK)]],
                                  msgs[b], gsems[b]).wait()
            pltpu.sync_copy(msgs[b], agg_sh.at[cols[b]], add=True)

        start(0, 0)
        if nch > 1:
            start(1, 1)

        def body(i, carry):
            j = 2 * i
            for b in range(2):
                finish(j + b, b)

                @pl.when(j + b + 2 < nch)
                def _():
                    start(j + b + 2, b)
            return carry

        lax.fori_loop(0, nch // 2, body, 0)
        if nch % 2:
            finish(nch - 1, (nch - 1) % 2)
        plsc.subcore_barrier()
        pltpu.sync_copy(agg_sh.at[pl.ds(r0, rp)],
                        out_hbm.at[cid, pl.ds(r0, rp)])
        if tail:
            @pl.when(sid == 0)
            def _():
                pltpu.sync_copy(agg_sh.at[pl.ds(_NS * rp, tail)],
                                out_hbm.at[cid, pl.ds(_NS * rp, tail)])

    return agg


_BR = 1000


def _tc_linear(x, w, b):
    N, D = x.shape

    def body(x_ref, w_ref, b_ref, o_ref):
        o_ref[...] = lax.dot_general(
            x_ref[...], w_ref[...], (((1,), (1,)), ((), ())),
            preferred_element_type=jnp.float32) + b_ref[...]

    return pl.pallas_call(
        body,
        grid=(N // _BR,),
        in_specs=[
            pl.BlockSpec((_BR, D), lambda i: (i, 0)),
            pl.BlockSpec((D, D), lambda i: (0, 0)),
            pl.BlockSpec((1, D), lambda i: (0, 0)),
        ],
        out_specs=pl.BlockSpec((_BR, D), lambda i: (i, 0)),
        out_shape=jax.ShapeDtypeStruct((N, D), jnp.float32),
    )(x, w, b)


def _tc_combine_linear(t, p, w, b):
    N, D = t.shape

    def body(t_ref, p_ref, w_ref, b_ref, o_ref):
        h = (t_ref[...] + p_ref[0] + p_ref[1]) * 0.5
        h = jnp.maximum(h, 0.0)
        o_ref[...] = lax.dot_general(
            h, w_ref[...], (((1,), (1,)), ((), ())),
            preferred_element_type=jnp.float32) + b_ref[...]

    return pl.pallas_call(
        body,
        grid=(N // _BR,),
        in_specs=[
            pl.BlockSpec((_BR, D), lambda i: (i, 0)),
            pl.BlockSpec((_NC, _BR, D), lambda i: (0, i, 0)),
            pl.BlockSpec((D, D), lambda i: (0, 0)),
            pl.BlockSpec((1, D), lambda i: (0, 0)),
        ],
        out_specs=pl.BlockSpec((_BR, D), lambda i: (i, 0)),
        out_shape=jax.ShapeDtypeStruct((N, D), jnp.float32),
    )(t, p, w, b)


def _tc_combine(t, p):
    N, D = t.shape

    def body(t_ref, p_ref, o_ref):
        o_ref[...] = (t_ref[...] + p_ref[0] + p_ref[1]) * 0.5

    return pl.pallas_call(
        body,
        grid=(N // _BR,),
        in_specs=[
            pl.BlockSpec((_BR, D), lambda i: (i, 0)),
            pl.BlockSpec((_NC, _BR, D), lambda i: (0, i, 0)),
        ],
        out_specs=pl.BlockSpec((_BR, D), lambda i: (i, 0)),
        out_shape=jax.ShapeDtypeStruct((N, D), jnp.float32),
    )(t, p)


def kernel(node_features, edge_index, weight0, bias0, weight1, bias1,
           hidden_dim):
    N, D = node_features.shape
    E = edge_index.shape[1]
    row = edge_index[0]
    col = edge_index[1]
    zeros = jnp.zeros((N, D), jnp.float32)

    sc_aggregate = _make_sc_aggregate(N, D, E)

    t1 = _tc_linear(node_features, weight0[0], bias0)
    p1 = sc_aggregate(t1, row, col, zeros)
    t2 = _tc_combine_linear(t1, p1, weight1[0], bias1)
    p2 = sc_aggregate(t2, row, col, zeros)
    return _tc_combine(t2, p2)


# speedup vs baseline: 13.2029x; 1.1437x over previous
import functools

import jax
import jax.numpy as jnp
from jax import lax
from jax.experimental import pallas as pl
from jax.experimental.pallas import tpu as pltpu
from jax.experimental.pallas import tpu_sc as plsc

_NC = 2
_NS = 16
_CHUNK = 80


def _make_sc_aggregate(N, D, E):
    NW = _NC * _NS
    ep = E // NW
    nch = ep // _CHUNK
    rp = (N // (8 * _NS)) * 8
    tail = N - _NS * rp

    mesh = plsc.VectorSubcoreMesh(core_axis_name="c", subcore_axis_name="s")

    @functools.partial(
        pl.kernel,
        mesh=mesh,
        out_type=jax.ShapeDtypeStruct((_NC, N, D), jnp.float32),
        scratch_types=[
            pltpu.VMEM((ep,), jnp.int32),
            pltpu.VMEM((_CHUNK,), jnp.int32),
            pltpu.VMEM((_CHUNK,), jnp.int32),
            pltpu.VMEM((_CHUNK,), jnp.int32),
            pltpu.VMEM((_CHUNK, D), jnp.float32),
            pltpu.VMEM((_CHUNK, D), jnp.float32),
            pltpu.VMEM((_CHUNK, D), jnp.float32),
            pltpu.VMEM_SHARED((N, D), jnp.float32),
            pltpu.SemaphoreType.DMA,
            pltpu.SemaphoreType.DMA,
            pltpu.SemaphoreType.DMA,
            pltpu.SemaphoreType.DMA,
            pltpu.SemaphoreType.DMA,
            pltpu.SemaphoreType.DMA,
            pltpu.SemaphoreType.DMA,
            pltpu.SemaphoreType.DMA,
            pltpu.SemaphoreType.DMA,
        ],
    )
    def agg(t_hbm, row_hbm, col_hbm, zero_hbm, out_hbm,
            row_all, col0, col1, col2, msg0, msg1, msg2, agg_sh,
            gsem0, gsem1, gsem2, csem0, csem1, csem2, ssem0, ssem1, ssem2):
        cid = lax.axis_index("c")
        sid = lax.axis_index("s")
        wid = cid * _NS + sid
        e0 = wid * ep
        cols = (col0, col1, col2)
        msgs = (msg0, msg1, msg2)
        gsems = (gsem0, gsem1, gsem2)
        csems = (csem0, csem1, csem2)
        ssems = (ssem0, ssem1, ssem2)

        r0 = sid * rp
        pltpu.sync_copy(zero_hbm.at[pl.ds(r0, rp)], agg_sh.at[pl.ds(r0, rp)])
        if tail:
            @pl.when(sid == 0)
            def _():
                pltpu.sync_copy(zero_hbm.at[pl.ds(_NS * rp, tail)],
                                agg_sh.at[pl.ds(_NS * rp, tail)])
        pltpu.sync_copy(row_hbm.at[pl.ds(e0, ep)], row_all)
        plsc.subcore_barrier()

        def start(j, b):
            pltpu.async_copy(col_hbm.at[pl.ds(e0 + j * _CHUNK, _CHUNK)],
                             cols[b], csems[b])
            pltpu.async_copy(t_hbm.at[row_all.at[pl.ds(j * _CHUNK, _CHUNK)]],
                             msgs[b], gsems[b])

        def wait_gather(j, b):
            pltpu.make_async_copy(col_hbm.at[pl.ds(e0 + j * _CHUNK, _CHUNK)],
                                  cols[b], csems[b]).wait()
            pltpu.make_async_copy(t_hbm.at[row_all.at[pl.ds(j * _CHUNK, _CHUNK)]],
                                  msgs[b], gsems[b]).wait()

        def drain_scatter(b):
            pltpu.make_async_copy(msgs[b], agg_sh.at[cols[b]], ssems[b]).wait()

        start(0, 0)
        if nch > 1:
            start(1, 1)

        def body(i, carry):
            j0 = 3 * i
            for b in range(3):
                j = j0 + b
                bn = (b + 2) % 3

                @pl.when(j < nch)
                def _():
                    wait_gather(j, b)
                    if b == 0:
                        @pl.when(j >= 1)
                        def _():
                            drain_scatter(bn)
                    else:
                        drain_scatter(bn)

                    @pl.when(j + 2 < nch)
                    def _():
                        start(j + 2, bn)
                    pltpu.async_copy(msgs[b], agg_sh.at[cols[b]], ssems[b],
                                     add=True)
            return carry

        lax.fori_loop(0, (nch + 2) // 3, body, 0)
        drain_scatter((nch - 1) % 3)
        plsc.subcore_barrier()
        pltpu.sync_copy(agg_sh.at[pl.ds(r0, rp)],
                        out_hbm.at[cid, pl.ds(r0, rp)])
        if tail:
            @pl.when(sid == 0)
            def _():
                pltpu.sync_copy(agg_sh.at[pl.ds(_NS * rp, tail)],
                                out_hbm.at[cid, pl.ds(_NS * rp, tail)])

    return agg


_BR = 1000


def _tc_linear(x, w, b):
    N, D = x.shape

    def body(x_ref, w_ref, b_ref, o_ref):
        o_ref[...] = lax.dot_general(
            x_ref[...], w_ref[...], (((1,), (1,)), ((), ())),
            preferred_element_type=jnp.float32) + b_ref[...]

    return pl.pallas_call(
        body,
        grid=(N // _BR,),
        in_specs=[
            pl.BlockSpec((_BR, D), lambda i: (i, 0)),
            pl.BlockSpec((D, D), lambda i: (0, 0)),
            pl.BlockSpec((1, D), lambda i: (0, 0)),
        ],
        out_specs=pl.BlockSpec((_BR, D), lambda i: (i, 0)),
        out_shape=jax.ShapeDtypeStruct((N, D), jnp.float32),
    )(x, w, b)


def _tc_combine_linear(t, p, w, b):
    N, D = t.shape

    def body(t_ref, p_ref, w_ref, b_ref, o_ref):
        h = (t_ref[...] + p_ref[0] + p_ref[1]) * 0.5
        h = jnp.maximum(h, 0.0)
        o_ref[...] = lax.dot_general(
            h, w_ref[...], (((1,), (1,)), ((), ())),
            preferred_element_type=jnp.float32) + b_ref[...]

    return pl.pallas_call(
        body,
        grid=(N // _BR,),
        in_specs=[
            pl.BlockSpec((_BR, D), lambda i: (i, 0)),
            pl.BlockSpec((_NC, _BR, D), lambda i: (0, i, 0)),
            pl.BlockSpec((D, D), lambda i: (0, 0)),
            pl.BlockSpec((1, D), lambda i: (0, 0)),
        ],
        out_specs=pl.BlockSpec((_BR, D), lambda i: (i, 0)),
        out_shape=jax.ShapeDtypeStruct((N, D), jnp.float32),
    )(t, p, w, b)


def _tc_combine(t, p):
    N, D = t.shape

    def body(t_ref, p_ref, o_ref):
        o_ref[...] = (t_ref[...] + p_ref[0] + p_ref[1]) * 0.5

    return pl.pallas_call(
        body,
        grid=(N // _BR,),
        in_specs=[
            pl.BlockSpec((_BR, D), lambda i: (i, 0)),
            pl.BlockSpec((_NC, _BR, D), lambda i: (0, i, 0)),
        ],
        out_specs=pl.BlockSpec((_BR, D), lambda i: (i, 0)),
        out_shape=jax.ShapeDtypeStruct((N, D), jnp.float32),
    )(t, p)


def kernel(node_features, edge_index, weight0, bias0, weight1, bias1,
           hidden_dim):
    N, D = node_features.shape
    E = edge_index.shape[1]
    row = edge_index[0]
    col = edge_index[1]
    zeros = jnp.zeros((N, D), jnp.float32)

    sc_aggregate = _make_sc_aggregate(N, D, E)

    t1 = _tc_linear(node_features, weight0[0], bias0)
    p1 = sc_aggregate(t1, row, col, zeros)
    t2 = _tc_combine_linear(t1, p1, weight1[0], bias1)
    p2 = sc_aggregate(t2, row, col, zeros)
    return _tc_combine(t2, p2)


# speedup vs baseline: 13.4645x; 1.0198x over previous
import functools

import jax
import jax.numpy as jnp
from jax import lax
from jax.experimental import pallas as pl
from jax.experimental.pallas import tpu as pltpu
from jax.experimental.pallas import tpu_sc as plsc

_NC = 2
_NS = 16
_CHUNK = 80


def _make_sc_aggregate(N, D, E):
    NW = _NC * _NS
    ep = E // NW
    nch = ep // _CHUNK
    rp = (N // (8 * _NS)) * 8
    tail = N - _NS * rp

    mesh = plsc.VectorSubcoreMesh(core_axis_name="c", subcore_axis_name="s")

    @functools.partial(
        pl.kernel,
        mesh=mesh,
        out_type=jax.ShapeDtypeStruct((_NC, N, D), jnp.float32),
        scratch_types=[
            pltpu.VMEM((ep,), jnp.int32),
            pltpu.VMEM((_CHUNK,), jnp.int32),
            pltpu.VMEM((_CHUNK,), jnp.int32),
            pltpu.VMEM((_CHUNK,), jnp.int32),
            pltpu.VMEM((_CHUNK, D), jnp.float32),
            pltpu.VMEM((_CHUNK, D), jnp.float32),
            pltpu.VMEM((_CHUNK, D), jnp.float32),
            pltpu.VMEM_SHARED((N, D), jnp.float32),
            pltpu.SemaphoreType.DMA,
            pltpu.SemaphoreType.DMA,
            pltpu.SemaphoreType.DMA,
            pltpu.SemaphoreType.DMA,
            pltpu.SemaphoreType.DMA,
            pltpu.SemaphoreType.DMA,
            pltpu.SemaphoreType.DMA,
            pltpu.SemaphoreType.DMA,
            pltpu.SemaphoreType.DMA,
        ],
    )
    def agg(t_hbm, row_hbm, col_hbm, zero_hbm, out_hbm,
            row_all, col0, col1, col2, msg0, msg1, msg2, agg_sh,
            gsem0, gsem1, gsem2, csem0, csem1, csem2, ssem0, ssem1, ssem2):
        cid = lax.axis_index("c")
        sid = lax.axis_index("s")
        wid = cid * _NS + sid
        e0 = wid * ep
        cols = (col0, col1, col2)
        msgs = (msg0, msg1, msg2)
        gsems = (gsem0, gsem1, gsem2)
        csems = (csem0, csem1, csem2)
        ssems = (ssem0, ssem1, ssem2)

        pltpu.sync_copy(row_hbm.at[pl.ds(e0, ep)], row_all)

        def start(j, b):
            pltpu.async_copy(col_hbm.at[pl.ds(e0 + j * _CHUNK, _CHUNK)],
                             cols[b], csems[b])
            pltpu.async_copy(t_hbm.at[row_all.at[pl.ds(j * _CHUNK, _CHUNK)]],
                             msgs[b], gsems[b])

        def wait_gather(j, b):
            pltpu.make_async_copy(col_hbm.at[pl.ds(e0 + j * _CHUNK, _CHUNK)],
                                  cols[b], csems[b]).wait()
            pltpu.make_async_copy(t_hbm.at[row_all.at[pl.ds(j * _CHUNK, _CHUNK)]],
                                  msgs[b], gsems[b]).wait()

        def drain_scatter(b):
            pltpu.make_async_copy(msgs[b], agg_sh.at[cols[b]], ssems[b]).wait()

        start(0, 0)
        if nch > 1:
            start(1, 1)

        r0 = sid * rp
        init = (t_hbm, zero_hbm)
        for c in range(_NC):
            @pl.when(cid == c)
            def _():
                pltpu.sync_copy(init[c].at[pl.ds(r0, rp)],
                                agg_sh.at[pl.ds(r0, rp)])
                if tail:
                    @pl.when(sid == 0)
                    def _():
                        pltpu.sync_copy(init[c].at[pl.ds(_NS * rp, tail)],
                                        agg_sh.at[pl.ds(_NS * rp, tail)])
        plsc.subcore_barrier()

        def body(i, carry):
            j0 = 3 * i
            for b in range(3):
                j = j0 + b
                bn = (b + 2) % 3

                @pl.when(j < nch)
                def _():
                    wait_gather(j, b)
                    if b == 0:
                        @pl.when(j >= 1)
                        def _():
                            drain_scatter(bn)
                    else:
                        drain_scatter(bn)

                    @pl.when(j + 2 < nch)
                    def _():
                        start(j + 2, bn)
                    pltpu.async_copy(msgs[b], agg_sh.at[cols[b]], ssems[b],
                                     add=True)
            return carry

        lax.fori_loop(0, (nch + 2) // 3, body, 0)
        drain_scatter((nch - 1) % 3)
        plsc.subcore_barrier()
        pltpu.sync_copy(agg_sh.at[pl.ds(r0, rp)],
                        out_hbm.at[cid, pl.ds(r0, rp)])
        if tail:
            @pl.when(sid == 0)
            def _():
                pltpu.sync_copy(agg_sh.at[pl.ds(_NS * rp, tail)],
                                out_hbm.at[cid, pl.ds(_NS * rp, tail)])

    return agg


_BR = 1000


def _tc_linear(x, w, b):
    N, D = x.shape

    def body(x_ref, w_ref, b_ref, o_ref):
        o_ref[...] = lax.dot_general(
            x_ref[...], w_ref[...] * 0.5, (((1,), (1,)), ((), ())),
            preferred_element_type=jnp.float32) + b_ref[...] * 0.5

    return pl.pallas_call(
        body,
        grid=(N // _BR,),
        in_specs=[
            pl.BlockSpec((_BR, D), lambda i: (i, 0)),
            pl.BlockSpec((D, D), lambda i: (0, 0)),
            pl.BlockSpec((1, D), lambda i: (0, 0)),
        ],
        out_specs=pl.BlockSpec((_BR, D), lambda i: (i, 0)),
        out_shape=jax.ShapeDtypeStruct((N, D), jnp.float32),
    )(x, w, b)


def _tc_combine_linear(p, w, b):
    N, D = p.shape[1], p.shape[2]

    def body(p_ref, w_ref, b_ref, o_ref):
        h = jnp.maximum(p_ref[0] + p_ref[1], 0.0)
        o_ref[...] = lax.dot_general(
            h, w_ref[...] * 0.5, (((1,), (1,)), ((), ())),
            preferred_element_type=jnp.float32) + b_ref[...] * 0.5

    return pl.pallas_call(
        body,
        grid=(N // _BR,),
        in_specs=[
            pl.BlockSpec((_NC, _BR, D), lambda i: (0, i, 0)),
            pl.BlockSpec((D, D), lambda i: (0, 0)),
            pl.BlockSpec((1, D), lambda i: (0, 0)),
        ],
        out_specs=pl.BlockSpec((_BR, D), lambda i: (i, 0)),
        out_shape=jax.ShapeDtypeStruct((N, D), jnp.float32),
    )(p, w, b)


def _tc_combine(p):
    N, D = p.shape[1], p.shape[2]

    def body(p_ref, o_ref):
        o_ref[...] = p_ref[0] + p_ref[1]

    return pl.pallas_call(
        body,
        grid=(N // _BR,),
        in_specs=[
            pl.BlockSpec((_NC, _BR, D), lambda i: (0, i, 0)),
        ],
        out_specs=pl.BlockSpec((_BR, D), lambda i: (i, 0)),
        out_shape=jax.ShapeDtypeStruct((N, D), jnp.float32),
    )(p)


def kernel(node_features, edge_index, weight0, bias0, weight1, bias1,
           hidden_dim):
    N, D = node_features.shape
    E = edge_index.shape[1]
    row = edge_index[0]
    col = edge_index[1]
    zeros = jnp.zeros((N, D), jnp.float32)

    sc_aggregate = _make_sc_aggregate(N, D, E)

    t1 = _tc_linear(node_features, weight0[0], bias0)
    p1 = sc_aggregate(t1, row, col, zeros)
    t2 = _tc_combine_linear(p1, weight1[0], bias1)
    p2 = sc_aggregate(t2, row, col, zeros)
    return _tc_combine(p2)


# speedup vs baseline: 14.2947x; 1.0617x over previous
import functools

import jax
import jax.numpy as jnp
from jax import lax
from jax.experimental import pallas as pl
from jax.experimental.pallas import tpu as pltpu
from jax.experimental.pallas import tpu_sc as plsc

_NC = 2
_NS = 16
_CHUNK = 80


def _make_sc_aggregate(N, D, E):
    NW = _NC * _NS
    ep = E // NW
    nch = ep // _CHUNK
    rp = (N // (8 * _NS)) * 8
    tail = N - _NS * rp

    mesh = plsc.VectorSubcoreMesh(core_axis_name="c", subcore_axis_name="s")

    @functools.partial(
        pl.kernel,
        mesh=mesh,
        out_type=jax.ShapeDtypeStruct((_NC, N, D), jnp.float32),
        scratch_types=[
            [pltpu.VMEM((_CHUNK,), jnp.int32) for _ in range(8)],
            [pltpu.VMEM((_CHUNK,), jnp.int32) for _ in range(8)],
            [pltpu.VMEM((_CHUNK, D), jnp.float32) for _ in range(4)],
            pltpu.VMEM_SHARED((N, D), jnp.float32),
            [pltpu.SemaphoreType.DMA for _ in range(8)],
            [pltpu.SemaphoreType.DMA for _ in range(4)],
            [pltpu.SemaphoreType.DMA for _ in range(4)],
        ],
    )
    def agg(t_hbm, row_hbm, col_hbm, zero_hbm, out_hbm,
            rowb, colb, msgs, agg_sh, isems, gsems, ssems):
        cid = lax.axis_index("c")
        sid = lax.axis_index("s")
        wid = cid * _NS + sid
        e0 = wid * ep

        def start_idx(j, ib):
            pltpu.async_copy(row_hbm.at[pl.ds(e0 + j * _CHUNK, _CHUNK)],
                             rowb[ib], isems[ib])
            pltpu.async_copy(col_hbm.at[pl.ds(e0 + j * _CHUNK, _CHUNK)],
                             colb[ib], isems[ib])

        def wait_idx(j, ib):
            pltpu.make_async_copy(row_hbm.at[pl.ds(e0 + j * _CHUNK, _CHUNK)],
                                  rowb[ib], isems[ib]).wait()
            pltpu.make_async_copy(col_hbm.at[pl.ds(e0 + j * _CHUNK, _CHUNK)],
                                  colb[ib], isems[ib]).wait()

        def start_gather(b, ib):
            pltpu.async_copy(t_hbm.at[rowb[ib]], msgs[b], gsems[b])

        def wait_gather(b, ib):
            pltpu.make_async_copy(t_hbm.at[rowb[ib]], msgs[b], gsems[b]).wait()

        def start_scatter(b, ib):
            pltpu.async_copy(msgs[b], agg_sh.at[colb[ib]], ssems[b], add=True)

        def drain_scatter(b, ib):
            pltpu.make_async_copy(msgs[b], agg_sh.at[colb[ib]], ssems[b]).wait()

        for k in range(5):
            start_idx(k, k)
        for k in range(3):
            wait_idx(k, k)
            start_gather(k, k)

        r0 = sid * rp
        init = (t_hbm, zero_hbm)
        for c in range(_NC):
            @pl.when(cid == c)
            def _():
                pltpu.sync_copy(init[c].at[pl.ds(r0, rp)],
                                agg_sh.at[pl.ds(r0, rp)])
                if tail:
                    @pl.when(sid == 0)
                    def _():
                        pltpu.sync_copy(init[c].at[pl.ds(_NS * rp, tail)],
                                        agg_sh.at[pl.ds(_NS * rp, tail)])
        plsc.subcore_barrier()

        def body(i, carry):
            j0 = 8 * i
            for p in range(8):
                j = j0 + p
                b = p % 4

                @pl.when(j < nch)
                def _():
                    wait_gather(b, p)
                    if p == 0:
                        @pl.when(j >= 1)
                        def _():
                            drain_scatter((b + 3) % 4, (p + 7) % 8)
                    else:
                        drain_scatter((b + 3) % 4, (p + 7) % 8)

                    @pl.when(j + 3 < nch)
                    def _():
                        wait_idx(j + 3, (p + 3) % 8)
                        start_gather((b + 3) % 4, (p + 3) % 8)

                    @pl.when(j + 5 < nch)
                    def _():
                        start_idx(j + 5, (p + 5) % 8)
                    start_scatter(b, p)
            return carry

        lax.fori_loop(0, (nch + 7) // 8, body, 0)
        drain_scatter((nch - 1) % 4, (nch - 1) % 8)
        plsc.subcore_barrier()
        pltpu.sync_copy(agg_sh.at[pl.ds(r0, rp)],
                        out_hbm.at[cid, pl.ds(r0, rp)])
        if tail:
            @pl.when(sid == 0)
            def _():
                pltpu.sync_copy(agg_sh.at[pl.ds(_NS * rp, tail)],
                                out_hbm.at[cid, pl.ds(_NS * rp, tail)])

    return agg


_BR = 1000


def _tc_linear(x, w, b):
    N, D = x.shape

    def body(x_ref, w_ref, b_ref, o_ref):
        o_ref[...] = lax.dot_general(
            x_ref[...], w_ref[...] * 0.5, (((1,), (1,)), ((), ())),
            preferred_element_type=jnp.float32) + b_ref[...] * 0.5

    return pl.pallas_call(
        body,
        grid=(N // _BR,),
        in_specs=[
            pl.BlockSpec((_BR, D), lambda i: (i, 0)),
            pl.BlockSpec((D, D), lambda i: (0, 0)),
            pl.BlockSpec((1, D), lambda i: (0, 0)),
        ],
        out_specs=pl.BlockSpec((_BR, D), lambda i: (i, 0)),
        out_shape=jax.ShapeDtypeStruct((N, D), jnp.float32),
    )(x, w, b)


def _tc_combine_linear(p, w, b):
    N, D = p.shape[1], p.shape[2]

    def body(p_ref, w_ref, b_ref, o_ref):
        h = jnp.maximum(p_ref[0] + p_ref[1], 0.0)
        o_ref[...] = lax.dot_general(
            h, w_ref[...] * 0.5, (((1,), (1,)), ((), ())),
            preferred_element_type=jnp.float32) + b_ref[...] * 0.5

    return pl.pallas_call(
        body,
        grid=(N // _BR,),
        in_specs=[
            pl.BlockSpec((_NC, _BR, D), lambda i: (0, i, 0)),
            pl.BlockSpec((D, D), lambda i: (0, 0)),
            pl.BlockSpec((1, D), lambda i: (0, 0)),
        ],
        out_specs=pl.BlockSpec((_BR, D), lambda i: (i, 0)),
        out_shape=jax.ShapeDtypeStruct((N, D), jnp.float32),
    )(p, w, b)


def _tc_combine(p):
    N, D = p.shape[1], p.shape[2]

    def body(p_ref, o_ref):
        o_ref[...] = p_ref[0] + p_ref[1]

    return pl.pallas_call(
        body,
        grid=(N // _BR,),
        in_specs=[
            pl.BlockSpec((_NC, _BR, D), lambda i: (0, i, 0)),
        ],
        out_specs=pl.BlockSpec((_BR, D), lambda i: (i, 0)),
        out_shape=jax.ShapeDtypeStruct((N, D), jnp.float32),
    )(p)


def kernel(node_features, edge_index, weight0, bias0, weight1, bias1,
           hidden_dim):
    N, D = node_features.shape
    E = edge_index.shape[1]
    row = edge_index[0]
    col = edge_index[1]
    zeros = jnp.zeros((N, D), jnp.float32)

    sc_aggregate = _make_sc_aggregate(N, D, E)

    t1 = _tc_linear(node_features, weight0[0], bias0)
    p1 = sc_aggregate(t1, row, col, zeros)
    t2 = _tc_combine_linear(p1, weight1[0], bias1)
    p2 = sc_aggregate(t2, row, col, zeros)
    return _tc_combine(p2)


# speedup vs baseline: 14.4899x; 1.0137x over previous
import functools

import jax
import jax.numpy as jnp
from jax import lax
from jax.experimental import pallas as pl
from jax.experimental.pallas import tpu as pltpu
from jax.experimental.pallas import tpu_sc as plsc

_NC = 2
_NS = 16
_CHUNK = 80


def _deg_pad(N):
    unit = 128 * _NS
    return ((N + unit - 1) // unit) * unit


def _make_sc_aggregate(N, D, E, with_deg):
    NW = _NC * _NS
    ep = E // NW
    nch = ep // _CHUNK
    rp = (N // (8 * _NS)) * 8
    tail = N - _NS * rp
    npad = _deg_pad(N)
    dp = npad // _NS

    mesh = plsc.VectorSubcoreMesh(core_axis_name="c", subcore_axis_name="s")

    out_type = jax.ShapeDtypeStruct((_NC, N, D), jnp.float32)
    scratch = [
        [pltpu.VMEM((_CHUNK,), jnp.int32) for _ in range(8)],
        [pltpu.VMEM((_CHUNK,), jnp.int32) for _ in range(8)],
        [pltpu.VMEM((_CHUNK, D), jnp.float32) for _ in range(4)],
        pltpu.VMEM_SHARED((N, D), jnp.float32),
        [pltpu.SemaphoreType.DMA for _ in range(8)],
        [pltpu.SemaphoreType.DMA for _ in range(4)],
        [pltpu.SemaphoreType.DMA for _ in range(4)],
    ]
    if with_deg:
        out_type = [out_type, jax.ShapeDtypeStruct((_NC, npad), jnp.float32)]
        scratch += [
            pltpu.VMEM((_CHUNK,), jnp.float32),
            pltpu.VMEM_SHARED((npad,), jnp.float32),
        ]

    @functools.partial(pl.kernel, mesh=mesh, out_type=out_type,
                       scratch_types=scratch)
    def agg(t_hbm, row_hbm, col_hbm, zero_hbm, zerod_hbm, *refs):
        if with_deg:
            (out_hbm, deg_hbm, rowb, colb, msgs, agg_sh,
             isems, gsems, ssems, ones_v, deg_sh) = refs
        else:
            out_hbm, rowb, colb, msgs, agg_sh, isems, gsems, ssems = refs
        cid = lax.axis_index("c")
        sid = lax.axis_index("s")
        wid = cid * _NS + sid
        e0 = wid * ep

        def start_idx(j, ib):
            pltpu.async_copy(row_hbm.at[pl.ds(e0 + j * _CHUNK, _CHUNK)],
                             rowb[ib], isems[ib])
            pltpu.async_copy(col_hbm.at[pl.ds(e0 + j * _CHUNK, _CHUNK)],
                             colb[ib], isems[ib])

        def wait_idx(j, ib):
            pltpu.make_async_copy(row_hbm.at[pl.ds(e0 + j * _CHUNK, _CHUNK)],
                                  rowb[ib], isems[ib]).wait()
            pltpu.make_async_copy(col_hbm.at[pl.ds(e0 + j * _CHUNK, _CHUNK)],
                                  colb[ib], isems[ib]).wait()

        def start_gather(b, ib):
            pltpu.async_copy(t_hbm.at[rowb[ib]], msgs[b], gsems[b])

        def wait_gather(b, ib):
            pltpu.make_async_copy(t_hbm.at[rowb[ib]], msgs[b], gsems[b]).wait()

        def start_scatter(b, ib):
            pltpu.async_copy(msgs[b], agg_sh.at[colb[ib]], ssems[b], add=True)
            if with_deg:
                pltpu.async_copy(ones_v, deg_sh.at[colb[ib]], ssems[b],
                                 add=True)

        def drain_scatter(b, ib):
            pltpu.make_async_copy(msgs[b], agg_sh.at[colb[ib]], ssems[b]).wait()
            if with_deg:
                pltpu.make_async_copy(ones_v, deg_sh.at[colb[ib]],
                                      ssems[b]).wait()

        for k in range(5):
            start_idx(k, k)
        for k in range(3):
            wait_idx(k, k)
            start_gather(k, k)

        r0 = sid * rp
        init = (t_hbm, zero_hbm)
        for c in range(_NC):
            @pl.when(cid == c)
            def _():
                pltpu.sync_copy(init[c].at[pl.ds(r0, rp)],
                                agg_sh.at[pl.ds(r0, rp)])
                if tail:
                    @pl.when(sid == 0)
                    def _():
                        pltpu.sync_copy(init[c].at[pl.ds(_NS * rp, tail)],
                                        agg_sh.at[pl.ds(_NS * rp, tail)])
        if with_deg:
            pltpu.sync_copy(zerod_hbm.at[pl.ds(sid * dp, dp)],
                            deg_sh.at[pl.ds(sid * dp, dp)])
            for k in range(_CHUNK // 16):
                ones_v[pl.ds(16 * k, 16)] = jnp.full((16,), 1.0, jnp.float32)
        plsc.subcore_barrier()

        def body(i, carry):
            j0 = 8 * i
            for p in range(8):
                j = j0 + p
                b = p % 4

                @pl.when(j < nch)
                def _():
                    wait_gather(b, p)
                    if p == 0:
                        @pl.when(j >= 1)
                        def _():
                            drain_scatter((b + 3) % 4, (p + 7) % 8)
                    else:
                        drain_scatter((b + 3) % 4, (p + 7) % 8)

                    @pl.when(j + 3 < nch)
                    def _():
                        wait_idx(j + 3, (p + 3) % 8)
                        start_gather((b + 3) % 4, (p + 3) % 8)

                    @pl.when(j + 5 < nch)
                    def _():
                        start_idx(j + 5, (p + 5) % 8)
                    start_scatter(b, p)
            return carry

        lax.fori_loop(0, (nch + 7) // 8, body, 0)
        drain_scatter((nch - 1) % 4, (nch - 1) % 8)
        plsc.subcore_barrier()
        pltpu.sync_copy(agg_sh.at[pl.ds(r0, rp)],
                        out_hbm.at[cid, pl.ds(r0, rp)])
        if tail:
            @pl.when(sid == 0)
            def _():
                pltpu.sync_copy(agg_sh.at[pl.ds(_NS * rp, tail)],
                                out_hbm.at[cid, pl.ds(_NS * rp, tail)])
        if with_deg:
            pltpu.sync_copy(deg_sh.at[pl.ds(sid * dp, dp)],
                            deg_hbm.at[cid, pl.ds(sid * dp, dp)])

    return agg


_BR = 1024


def _tc_transform(p, dg, w, b, relu):
    N, D = p.shape[1], p.shape[2]

    def body(p_ref, dg_ref, w_ref, b_ref, o_ref):
        s = p_ref[0] + p_ref[1]
        scale = (1.0 + dg_ref[0] + dg_ref[1])[:, None]
        o = lax.dot_general(
            s, w_ref[...] * 0.5, (((1,), (1,)), ((), ())),
            preferred_element_type=jnp.float32) + scale * (b_ref[...] * 0.5)
        if relu:
            o = jnp.maximum(o, 0.0)
        o_ref[...] = o

    return pl.pallas_call(
        body,
        grid=((N + _BR - 1) // _BR,),
        in_specs=[
            pl.BlockSpec((_NC, _BR, D), lambda i: (0, i, 0)),
            pl.BlockSpec((_NC, _BR), lambda i: (0, i)),
            pl.BlockSpec((D, D), lambda i: (0, 0)),
            pl.BlockSpec((1, D), lambda i: (0, 0)),
        ],
        out_specs=pl.BlockSpec((_BR, D), lambda i: (i, 0)),
        out_shape=jax.ShapeDtypeStruct((N, D), jnp.float32),
    )(p, dg, w, b)


def kernel(node_features, edge_index, weight0, bias0, weight1, bias1,
           hidden_dim):
    N, D = node_features.shape
    E = edge_index.shape[1]
    row = edge_index[0]
    col = edge_index[1]
    zeros = jnp.zeros((N, D), jnp.float32)
    zerod = jnp.zeros((_deg_pad(N),), jnp.float32)

    sc_agg_deg = _make_sc_aggregate(N, D, E, True)
    sc_agg = _make_sc_aggregate(N, D, E, False)

    p1, deg = sc_agg_deg(node_features, row, col, zeros, zerod)
    h = _tc_transform(p1, deg, weight0[0], bias0, True)
    p2 = sc_agg(h, row, col, zeros, zerod)
    return _tc_transform(p2, deg, weight1[0], bias1, False)


# speedup vs baseline: 15.1873x; 1.0481x over previous
import functools

import jax
import jax.numpy as jnp
from jax import lax
from jax.experimental import pallas as pl
from jax.experimental.pallas import tpu as pltpu
from jax.experimental.pallas import tpu_sc as plsc

_NC = 2
_NS = 16
_CHUNK = 80


def _deg_pad(N):
    unit = 128 * _NS
    return ((N + unit - 1) // unit) * unit


def _make_sc_aggregate(N, D, E, with_deg):
    NW = _NC * _NS
    ep = E // NW
    nch = ep // _CHUNK
    rp = (N // (8 * _NS)) * 8
    tail = N - _NS * rp
    npad = _deg_pad(N)
    dp = npad // _NS

    mesh = plsc.VectorSubcoreMesh(core_axis_name="c", subcore_axis_name="s")

    out_type = jax.ShapeDtypeStruct((_NC, N, D), jnp.float32)
    scratch = [
        [pltpu.VMEM((_CHUNK,), jnp.int32) for _ in range(8)],
        [pltpu.VMEM((_CHUNK,), jnp.int32) for _ in range(8)],
        [pltpu.VMEM((_CHUNK, D), jnp.float32) for _ in range(4)],
        pltpu.VMEM_SHARED((N, D), jnp.float32),
        [pltpu.SemaphoreType.DMA for _ in range(8)],
        [pltpu.SemaphoreType.DMA for _ in range(4)],
        [pltpu.SemaphoreType.DMA for _ in range(4)],
    ]
    if with_deg:
        out_type = [out_type, jax.ShapeDtypeStruct((_NC, npad), jnp.float32)]
        scratch += [
            pltpu.VMEM((_CHUNK,), jnp.float32),
            pltpu.VMEM_SHARED((npad,), jnp.float32),
        ]

    @functools.partial(pl.kernel, mesh=mesh, out_type=out_type,
                       scratch_types=scratch)
    def agg(t_hbm, edge_hbm, zero_hbm, zerod_hbm, *refs):
        if with_deg:
            (out_hbm, deg_hbm, rowb, colb, msgs, agg_sh,
             isems, gsems, ssems, ones_v, deg_sh) = refs
        else:
            out_hbm, rowb, colb, msgs, agg_sh, isems, gsems, ssems = refs
        cid = lax.axis_index("c")
        sid = lax.axis_index("s")
        wid = cid * _NS + sid
        e0 = wid * ep
        c0 = E + wid * ep

        def start_idx(j, ib):
            pltpu.async_copy(edge_hbm.at[pl.ds(e0 + j * _CHUNK, _CHUNK)],
                             rowb[ib], isems[ib])
            pltpu.async_copy(edge_hbm.at[pl.ds(c0 + j * _CHUNK, _CHUNK)],
                             colb[ib], isems[ib])

        def wait_idx(j, ib):
            pltpu.make_async_copy(edge_hbm.at[pl.ds(e0 + j * _CHUNK, _CHUNK)],
                                  rowb[ib], isems[ib]).wait()
            pltpu.make_async_copy(edge_hbm.at[pl.ds(c0 + j * _CHUNK, _CHUNK)],
                                  colb[ib], isems[ib]).wait()

        def start_gather(b, ib):
            pltpu.async_copy(t_hbm.at[rowb[ib]], msgs[b], gsems[b])

        def wait_gather(b, ib):
            pltpu.make_async_copy(t_hbm.at[rowb[ib]], msgs[b], gsems[b]).wait()

        def start_scatter(b, ib):
            pltpu.async_copy(msgs[b], agg_sh.at[colb[ib]], ssems[b], add=True)
            if with_deg:
                pltpu.async_copy(ones_v, deg_sh.at[colb[ib]], ssems[b],
                                 add=True)

        def drain_scatter(b, ib):
            pltpu.make_async_copy(msgs[b], agg_sh.at[colb[ib]], ssems[b]).wait()
            if with_deg:
                pltpu.make_async_copy(ones_v, deg_sh.at[colb[ib]],
                                      ssems[b]).wait()

        for k in range(5):
            start_idx(k, k)
        for k in range(3):
            wait_idx(k, k)
            start_gather(k, k)

        r0 = sid * rp
        init = (t_hbm, zero_hbm)
        for c in range(_NC):
            @pl.when(cid == c)
            def _():
                pltpu.sync_copy(init[c].at[pl.ds(r0, rp)],
                                agg_sh.at[pl.ds(r0, rp)])
                if tail:
                    @pl.when(sid == 0)
                    def _():
                        pltpu.sync_copy(init[c].at[pl.ds(_NS * rp, tail)],
                                        agg_sh.at[pl.ds(_NS * rp, tail)])
        if with_deg:
            pltpu.sync_copy(zerod_hbm.at[pl.ds(sid * dp, dp)],
                            deg_sh.at[pl.ds(sid * dp, dp)])
            for k in range(_CHUNK // 16):
                ones_v[pl.ds(16 * k, 16)] = jnp.full((16,), 1.0, jnp.float32)
        plsc.subcore_barrier()

        def body(i, carry):
            j0 = 8 * i
            for p in range(8):
                j = j0 + p
                b = p % 4

                @pl.when(j < nch)
                def _():
                    wait_gather(b, p)
                    if p == 0:
                        @pl.when(j >= 1)
                        def _():
                            drain_scatter((b + 3) % 4, (p + 7) % 8)
                    else:
                        drain_scatter((b + 3) % 4, (p + 7) % 8)

                    @pl.when(j + 3 < nch)
                    def _():
                        wait_idx(j + 3, (p + 3) % 8)
                        start_gather((b + 3) % 4, (p + 3) % 8)

                    @pl.when(j + 5 < nch)
                    def _():
                        start_idx(j + 5, (p + 5) % 8)
                    start_scatter(b, p)
            return carry

        lax.fori_loop(0, (nch + 7) // 8, body, 0)
        drain_scatter((nch - 1) % 4, (nch - 1) % 8)
        plsc.subcore_barrier()
        pltpu.sync_copy(agg_sh.at[pl.ds(r0, rp)],
                        out_hbm.at[cid, pl.ds(r0, rp)])
        if tail:
            @pl.when(sid == 0)
            def _():
                pltpu.sync_copy(agg_sh.at[pl.ds(_NS * rp, tail)],
                                out_hbm.at[cid, pl.ds(_NS * rp, tail)])
        if with_deg:
            pltpu.sync_copy(deg_sh.at[pl.ds(sid * dp, dp)],
                            deg_hbm.at[cid, pl.ds(sid * dp, dp)])

    return agg


_BR = 1024


def _tc_transform(p, dg, w, b, relu):
    N, D = p.shape[1], p.shape[2]

    def body(p_ref, dg_ref, w_ref, b_ref, o_ref):
        s = p_ref[0] + p_ref[1]
        scale = (1.0 + dg_ref[0] + dg_ref[1])[:, None]
        o = lax.dot_general(
            s, w_ref[...] * 0.5, (((1,), (1,)), ((), ())),
            preferred_element_type=jnp.float32) + scale * (b_ref[...] * 0.5)
        if relu:
            o = jnp.maximum(o, 0.0)
        o_ref[...] = o

    return pl.pallas_call(
        body,
        grid=((N + _BR - 1) // _BR,),
        in_specs=[
            pl.BlockSpec((_NC, _BR, D), lambda i: (0, i, 0)),
            pl.BlockSpec((_NC, _BR), lambda i: (0, i)),
            pl.BlockSpec((D, D), lambda i: (0, 0)),
            pl.BlockSpec((1, D), lambda i: (0, 0)),
        ],
        out_specs=pl.BlockSpec((_BR, D), lambda i: (i, 0)),
        out_shape=jax.ShapeDtypeStruct((N, D), jnp.float32),
    )(p, dg, w, b)


def kernel(node_features, edge_index, weight0, bias0, weight1, bias1,
           hidden_dim):
    N, D = node_features.shape
    E = edge_index.shape[1]
    edges = edge_index.reshape(2 * E)
    zeros = jnp.zeros((N, D), jnp.float32)
    zerod = jnp.zeros((_deg_pad(N),), jnp.float32)

    sc_agg_deg = _make_sc_aggregate(N, D, E, True)
    sc_agg = _make_sc_aggregate(N, D, E, False)

    p1, deg = sc_agg_deg(node_features, edges, zeros, zerod)
    h = _tc_transform(p1, deg, weight0[0], bias0, True)
    p2 = sc_agg(h, edges, zeros, zerod)
    return _tc_transform(p2, deg, weight1[0], bias1, False)


# speedup vs baseline: 15.2981x; 1.0073x over previous
import functools

import jax
import jax.numpy as jnp
from jax import lax
from jax.experimental import pallas as pl
from jax.experimental.pallas import tpu as pltpu
from jax.experimental.pallas import tpu_sc as plsc

_NC = 2
_NS = 16
_CHUNK = 80


def _deg_pad(N):
    unit = 128 * _NS
    return ((N + unit - 1) // unit) * unit


def _make_sc_aggregate(N, D, E, with_deg):
    NW = _NC * _NS
    ep = E // NW
    nch = ep // _CHUNK
    rp = (N // (8 * _NS)) * 8
    tail = N - _NS * rp
    npad = _deg_pad(N)
    dp = npad // _NS

    mesh = plsc.VectorSubcoreMesh(core_axis_name="c", subcore_axis_name="s")

    out_type = jax.ShapeDtypeStruct((_NC, N, D), jnp.float32)
    scratch = [
        [pltpu.VMEM((_CHUNK,), jnp.int32) for _ in range(8)],
        [pltpu.VMEM((_CHUNK,), jnp.int32) for _ in range(8)],
        [pltpu.VMEM((_CHUNK, D), jnp.float32) for _ in range(4)],
        pltpu.VMEM_SHARED((N, D), jnp.float32),
        [pltpu.SemaphoreType.DMA for _ in range(8)],
        [pltpu.SemaphoreType.DMA for _ in range(4)],
        [pltpu.SemaphoreType.DMA for _ in range(4)],
    ]
    if with_deg:
        out_type = [out_type, jax.ShapeDtypeStruct((_NC, npad), jnp.float32)]
        scratch += [
            pltpu.VMEM((_CHUNK,), jnp.float32),
            pltpu.VMEM((128,), jnp.float32),
            pltpu.VMEM_SHARED((npad,), jnp.float32),
        ]

    @functools.partial(pl.kernel, mesh=mesh, out_type=out_type,
                       scratch_types=scratch)
    def agg(t_hbm, edge_hbm, *refs):
        if with_deg:
            (out_hbm, deg_hbm, rowb, colb, msgs, agg_sh,
             isems, gsems, ssems, ones_v, zero_v, deg_sh) = refs
        else:
            out_hbm, rowb, colb, msgs, agg_sh, isems, gsems, ssems = refs
        cid = lax.axis_index("c")
        sid = lax.axis_index("s")
        wid = cid * _NS + sid
        e0 = wid * ep
        c0 = E + wid * ep

        def start_idx(j, ib):
            pltpu.async_copy(edge_hbm.at[pl.ds(e0 + j * _CHUNK, _CHUNK)],
                             rowb[ib], isems[ib])
            pltpu.async_copy(edge_hbm.at[pl.ds(c0 + j * _CHUNK, _CHUNK)],
                             colb[ib], isems[ib])

        def wait_idx(j, ib):
            pltpu.make_async_copy(edge_hbm.at[pl.ds(e0 + j * _CHUNK, _CHUNK)],
                                  rowb[ib], isems[ib]).wait()
            pltpu.make_async_copy(edge_hbm.at[pl.ds(c0 + j * _CHUNK, _CHUNK)],
                                  colb[ib], isems[ib]).wait()

        def start_gather(b, ib):
            pltpu.async_copy(t_hbm.at[rowb[ib]], msgs[b], gsems[b])

        def wait_gather(b, ib):
            pltpu.make_async_copy(t_hbm.at[rowb[ib]], msgs[b], gsems[b]).wait()

        def start_scatter(b, ib):
            pltpu.async_copy(msgs[b], agg_sh.at[colb[ib]], ssems[b], add=True)
            if with_deg:
                pltpu.async_copy(ones_v, deg_sh.at[colb[ib]], ssems[b],
                                 add=True)

        def drain_scatter(b, ib):
            pltpu.make_async_copy(msgs[b], agg_sh.at[colb[ib]], ssems[b]).wait()
            if with_deg:
                pltpu.make_async_copy(ones_v, deg_sh.at[colb[ib]],
                                      ssems[b]).wait()

        for k in range(5):
            start_idx(k, k)
        for k in range(3):
            wait_idx(k, k)
            start_gather(k, k)

        r0 = sid * rp
        pltpu.sync_copy(t_hbm.at[pl.ds(r0, rp)], agg_sh.at[pl.ds(r0, rp)])
        if tail:
            @pl.when(sid == 0)
            def _():
                pltpu.sync_copy(t_hbm.at[pl.ds(_NS * rp, tail)],
                                agg_sh.at[pl.ds(_NS * rp, tail)])
        if with_deg:
            for k in range(_CHUNK // 16):
                ones_v[pl.ds(16 * k, 16)] = jnp.full((16,), 1.0, jnp.float32)
            for k in range(8):
                zero_v[pl.ds(16 * k, 16)] = jnp.zeros((16,), jnp.float32)
            for m in range(dp // 128):
                pltpu.sync_copy(zero_v,
                                deg_sh.at[pl.ds(sid * dp + 128 * m, 128)])
        plsc.subcore_barrier()

        def body(i, carry):
            j0 = 8 * i
            for p in range(8):
                j = j0 + p
                b = p % 4

                @pl.when(j < nch)
                def _():
                    wait_gather(b, p)
                    if p == 0:
                        @pl.when(j >= 1)
                        def _():
                            drain_scatter((b + 3) % 4, (p + 7) % 8)
                    else:
                        drain_scatter((b + 3) % 4, (p + 7) % 8)

                    @pl.when(j + 3 < nch)
                    def _():
                        wait_idx(j + 3, (p + 3) % 8)
                        start_gather((b + 3) % 4, (p + 3) % 8)

                    @pl.when(j + 5 < nch)
                    def _():
                        start_idx(j + 5, (p + 5) % 8)
                    start_scatter(b, p)
            return carry

        lax.fori_loop(0, (nch + 7) // 8, body, 0)
        drain_scatter((nch - 1) % 4, (nch - 1) % 8)
        plsc.subcore_barrier()
        pltpu.sync_copy(agg_sh.at[pl.ds(r0, rp)],
                        out_hbm.at[cid, pl.ds(r0, rp)])
        if tail:
            @pl.when(sid == 0)
            def _():
                pltpu.sync_copy(agg_sh.at[pl.ds(_NS * rp, tail)],
                                out_hbm.at[cid, pl.ds(_NS * rp, tail)])
        if with_deg:
            pltpu.sync_copy(deg_sh.at[pl.ds(sid * dp, dp)],
                            deg_hbm.at[cid, pl.ds(sid * dp, dp)])

    return agg


_BR = 1024


def _tc_transform(p, t, dg, w, b, relu):
    N, D = p.shape[1], p.shape[2]

    def body(p_ref, t_ref, dg_ref, w_ref, b_ref, o_ref):
        s = p_ref[0] + p_ref[1] - t_ref[...]
        scale = (1.0 + dg_ref[0] + dg_ref[1])[:, None]
        o = lax.dot_general(
            s, w_ref[...] * 0.5, (((1,), (1,)), ((), ())),
            preferred_element_type=jnp.float32) + scale * (b_ref[...] * 0.5)
        if relu:
            o = jnp.maximum(o, 0.0)
        o_ref[...] = o

    return pl.pallas_call(
        body,
        grid=((N + _BR - 1) // _BR,),
        in_specs=[
            pl.BlockSpec((_NC, _BR, D), lambda i: (0, i, 0)),
            pl.BlockSpec((_BR, D), lambda i: (i, 0)),
            pl.BlockSpec((_NC, _BR), lambda i: (0, i)),
            pl.BlockSpec((D, D), lambda i: (0, 0)),
            pl.BlockSpec((1, D), lambda i: (0, 0)),
        ],
        out_specs=pl.BlockSpec((_BR, D), lambda i: (i, 0)),
        out_shape=jax.ShapeDtypeStruct((N, D), jnp.float32),
    )(p, t, dg, w, b)


def kernel(node_features, edge_index, weight0, bias0, weight1, bias1,
           hidden_dim):
    N, D = node_features.shape
    E = edge_index.shape[1]
    edges = edge_index.reshape(2 * E)

    sc_agg_deg = _make_sc_aggregate(N, D, E, True)
    sc_agg = _make_sc_aggregate(N, D, E, False)

    p1, deg = sc_agg_deg(node_features, edges)
    h = _tc_transform(p1, node_features, deg, weight0[0], bias0, True)
    p2 = sc_agg(h, edges)
    return _tc_transform(p2, h, deg, weight1[0], bias1, False)
